# baseline (speedup 1.0000x reference)
import jax
import jax.numpy as jnp
from jax import lax
from jax.experimental import pallas as pl
from jax.experimental.pallas import tpu as pltpu

_L_LIPS = 8
_L_EXP = 8
_L_REST = 5
_L_ROT = 7
_OFF_LIPS = 0
_OFF_EXP = _OFF_LIPS + _L_LIPS ** 5
_OFF_REST = _OFF_EXP + _L_EXP ** 5
_OFF_ROT = _OFF_REST + _L_REST ** 6

_F = 205
_DQ = 20
_SB = 16
_TB = 256


def _u(n, f32):
    r = lax.broadcasted_iota(jnp.int32, (n * _SB, n), 0) // _SB
    k = lax.broadcasted_iota(jnp.int32, (n * _SB, n), 1)
    return jnp.where(r == k, jnp.float32(1.0), jnp.float32(0.0)).astype(f32)


def _kron_i(m, f32):
    rr, cc = m.shape[0] * _SB, m.shape[1] * _SB
    tmp = lax.dot_general(m, _u(m.shape[1], f32), (((1,), (1,)), ((), ())),
                          preferred_element_type=f32)
    exp = jnp.dot(_u(m.shape[0], f32), tmp,
                  preferred_element_type=f32)
    bmask = (lax.broadcasted_iota(jnp.int32, (rr, cc), 0) & (_SB - 1)) == \
            (lax.broadcasted_iota(jnp.int32, (rr, cc), 1) & (_SB - 1))
    return jnp.where(bmask, exp, jnp.float32(0.0))


def _body(x_ref, wl_ref, we_ref, wr_ref, wo_ref, dl_ref, de_ref, dr_ref,
          do_ref, half_ref, pt_ref, c_ref, out_ref, codes_ref,
          wa_s, wohi_s, wolo_s, pa_s, half_s, ca_s):
    f32 = jnp.float32

    @pl.when(pl.program_id(0) == 0)
    def _assemble():
        z = lambda r, c: jnp.zeros((r, c), f32)
        r_lips = jnp.concatenate([z(5, 12), wl_ref[...], z(5, 130)], axis=1)
        r_exp = jnp.concatenate([z(5, 12), we_ref[...], z(5, 130)], axis=1)
        r_rest = jnp.concatenate([z(6, 75), wr_ref[...]], axis=1)
        r_rot = jnp.concatenate([wo_ref[...], z(4, 193)], axis=1)
        wallt = jnp.concatenate([r_lips, r_exp, r_rest, r_rot], axis=0)
        dl, de, dr, do = dl_ref[...], de_ref[...], dr_ref[...], do_ref[...]
        r2_lips = jnp.concatenate([z(5, 60), dl, z(5, 130)], axis=1)
        r2_exp = jnp.concatenate([z(5, 12), de, z(5, 145)], axis=1)
        r2_rest = jnp.concatenate(
            [z(6, 9), dr[:, 0:3], z(6, 63), dr[:, 3:66], z(6, 1),
             dr[:, 66:69], dr[:, 69:132]], axis=1)
        r2_rot = jnp.concatenate(
            [do[:, 0:9], z(4, 129), do[:, 9:10], z(4, 66)], axis=1)
        wout = jnp.concatenate([r2_lips, r2_exp, r2_rest, r2_rot], axis=0)

        wa_s[...] = _kron_i(wallt, f32)
        tmp = lax.dot_general(_u(_F, f32), wout, (((1,), (1,)), ((), ())),
                              preferred_element_type=f32)
        woa = lax.dot_general(tmp, _u(_DQ, f32), (((1,), (1,)), ((), ())),
                              preferred_element_type=f32)
        rr = lax.broadcasted_iota(jnp.int32, (_F * _SB, _DQ * _SB), 0)
        cc = lax.broadcasted_iota(jnp.int32, (_F * _SB, _DQ * _SB), 1)
        woa = jnp.where((rr & (_SB - 1)) == (cc & (_SB - 1)), woa,
                        jnp.float32(0.0))
        wohi = woa.astype(jnp.bfloat16)
        wohi_s[...] = wohi
        wolo_s[...] = (woa - wohi.astype(f32)).astype(jnp.bfloat16)
        pa_s[...] = _kron_i(pt_ref[...], f32)
        half_s[...] = jnp.dot(_u(_DQ, f32), half_ref[...],
                              preferred_element_type=f32)
        ca_s[...] = jnp.dot(_u(4, f32), c_ref[...],
                            preferred_element_type=f32)

    xb = x_ref[...]
    zt = jnp.dot(wa_s[...], xb, preferred_element_type=f32)
    q = jnp.round(half_s[...] * jnp.tanh(zt))
    qb = q.astype(jnp.bfloat16)
    out_ref[...] = (jnp.dot(wohi_s[...], qb, preferred_element_type=f32)
                    + jnp.dot(wolo_s[...], qb, preferred_element_type=f32))
    codes_t = jnp.dot(pa_s[...], q, preferred_element_type=f32)
    codes_ref[...] = (codes_t + ca_s[...]).astype(jnp.int32)


def kernel(x, W_enc_lips, W_enc_exp, W_enc_rest, W_enc_rot,
           W_dec_lips, W_dec_exp, W_dec_rest, W_dec_rot):
    B, T, F = x.shape
    f32 = jnp.float32

    xa = x.transpose(2, 0, 1).reshape(F * B, T)
    wlt, wet = W_enc_lips.T, W_enc_exp.T
    wrt, wot = W_enc_rest.T, W_enc_rot.T

    half = jnp.array([[3.5]] * 10 + [[2.0]] * 6 + [[3.0]] * 4, f32)
    pt = jnp.array(
        [[float(_L_LIPS ** i) for i in range(5)] + [0.0] * 15,
         [0.0] * 5 + [float(_L_EXP ** i) for i in range(5)] + [0.0] * 10,
         [0.0] * 10 + [float(_L_REST ** i) for i in range(6)] + [0.0] * 4,
         [0.0] * 16 + [float(_L_ROT ** i) for i in range(4)]], f32)
    c_vec = jnp.array(
        [[_OFF_LIPS + 3.0 * sum(_L_LIPS ** i for i in range(5))],
         [_OFF_EXP + 3.0 * sum(_L_EXP ** i for i in range(5))],
         [_OFF_REST + 2.0 * sum(_L_REST ** i for i in range(6))],
         [_OFF_ROT + 3.0 * sum(_L_ROT ** i for i in range(4))]], f32)

    full = lambda shape: pl.BlockSpec(shape, lambda b: tuple(0 for _ in shape))
    outa, codesa = pl.pallas_call(
        _body,
        grid=(T // _TB,),
        in_specs=[
            pl.BlockSpec((F * B, _TB), lambda b: (0, b)),
            full((5, 63)), full((5, 63)), full((6, 130)), full((4, 12)),
            full((5, 15)), full((5, 48)), full((6, 132)), full((4, 10)),
            full((_DQ, 1)), full((4, _DQ)), full((4, 1)),
        ],
        out_specs=[
            pl.BlockSpec((F * B, _TB), lambda b: (0, b)),
            pl.BlockSpec((4 * B, _TB), lambda b: (0, b)),
        ],
        out_shape=[
            jax.ShapeDtypeStruct((F * B, T), f32),
            jax.ShapeDtypeStruct((4 * B, T), jnp.int32),
        ],
        scratch_shapes=[
            pltpu.VMEM((_DQ * B, F * B), f32),
            pltpu.VMEM((F * B, _DQ * B), jnp.bfloat16),
            pltpu.VMEM((F * B, _DQ * B), jnp.bfloat16),
            pltpu.VMEM((4 * B, _DQ * B), f32),
            pltpu.VMEM((_DQ * B, 1), f32),
            pltpu.VMEM((4 * B, 1), f32),
        ],
        compiler_params=pltpu.CompilerParams(
            dimension_semantics=("arbitrary",),
        ),
    )(xa, wlt, wet, wrt, wot, W_dec_lips, W_dec_exp, W_dec_rest, W_dec_rot,
      half, pt, c_vec)

    out = outa.reshape(F, B, T).transpose(1, 2, 0)
    codes = codesa.reshape(4, B, T)
    return out, codes


# speedup vs baseline: 1.4922x; 1.4922x over previous
import jax
import jax.numpy as jnp
from jax import lax
from jax.experimental import pallas as pl
from jax.experimental.pallas import tpu as pltpu

_L_LIPS = 8
_L_EXP = 8
_L_REST = 5
_L_ROT = 7
_OFF_LIPS = 0
_OFF_EXP = _OFF_LIPS + _L_LIPS ** 5
_OFF_REST = _OFF_EXP + _L_EXP ** 5
_OFF_ROT = _OFF_REST + _L_REST ** 6

_F = 205
_DQ = 20
_SB = 16
_TB = 512


def _u(n, f32):
    r = lax.broadcasted_iota(jnp.int32, (n * _SB, n), 0) // _SB
    k = lax.broadcasted_iota(jnp.int32, (n * _SB, n), 1)
    return jnp.where(r == k, jnp.float32(1.0), jnp.float32(0.0)).astype(f32)


def _kron_i(m, f32):
    rr, cc = m.shape[0] * _SB, m.shape[1] * _SB
    tmp = lax.dot_general(m, _u(m.shape[1], f32), (((1,), (1,)), ((), ())),
                          preferred_element_type=f32)
    exp = jnp.dot(_u(m.shape[0], f32), tmp,
                  preferred_element_type=f32)
    bmask = (lax.broadcasted_iota(jnp.int32, (rr, cc), 0) & (_SB - 1)) == \
            (lax.broadcasted_iota(jnp.int32, (rr, cc), 1) & (_SB - 1))
    return jnp.where(bmask, exp, jnp.float32(0.0))


def _body(x_ref, wl_ref, we_ref, wr_ref, wo_ref, dl_ref, de_ref, dr_ref,
          do_ref, half_ref, pt_ref, c_ref, out_ref, codes_ref,
          wa_s, wo_s, pa_s, half_s, ca_s):
    f32 = jnp.float32

    @pl.when(pl.program_id(0) == 0)
    def _assemble():
        z = lambda r, c: jnp.zeros((r, c), f32)
        r_lips = jnp.concatenate([z(5, 12), wl_ref[...], z(5, 130)], axis=1)
        r_exp = jnp.concatenate([z(5, 12), we_ref[...], z(5, 130)], axis=1)
        r_rest = jnp.concatenate([z(6, 75), wr_ref[...]], axis=1)
        r_rot = jnp.concatenate([wo_ref[...], z(4, 193)], axis=1)
        wallt = jnp.concatenate([r_lips, r_exp, r_rest, r_rot], axis=0)
        dl, de, dr, do = dl_ref[...], de_ref[...], dr_ref[...], do_ref[...]
        r2_lips = jnp.concatenate([z(5, 60), dl, z(5, 130)], axis=1)
        r2_exp = jnp.concatenate([z(5, 12), de, z(5, 145)], axis=1)
        r2_rest = jnp.concatenate(
            [z(6, 9), dr[:, 0:3], z(6, 63), dr[:, 3:66], z(6, 1),
             dr[:, 66:69], dr[:, 69:132]], axis=1)
        r2_rot = jnp.concatenate(
            [do[:, 0:9], z(4, 129), do[:, 9:10], z(4, 66)], axis=1)
        wout = jnp.concatenate([r2_lips, r2_exp, r2_rest, r2_rot], axis=0)

        wa_s[...] = _kron_i(wallt, f32)
        tmp = lax.dot_general(_u(_F, f32), wout, (((1,), (1,)), ((), ())),
                              preferred_element_type=f32)
        woa = lax.dot_general(tmp, _u(_DQ, f32), (((1,), (1,)), ((), ())),
                              preferred_element_type=f32)
        rr = lax.broadcasted_iota(jnp.int32, (_F * _SB, _DQ * _SB), 0)
        cc = lax.broadcasted_iota(jnp.int32, (_F * _SB, _DQ * _SB), 1)
        woa = jnp.where((rr & (_SB - 1)) == (cc & (_SB - 1)), woa,
                        jnp.float32(0.0))
        wo_s[...] = woa
        pa_s[...] = _kron_i(pt_ref[...], f32)
        half_s[...] = jnp.dot(_u(_DQ, f32), half_ref[...],
                              preferred_element_type=f32)
        ca_s[...] = jnp.dot(_u(4, f32), c_ref[...],
                            preferred_element_type=f32)

    xb = x_ref[...]
    zt = jnp.dot(wa_s[...], xb, preferred_element_type=f32)
    q = jnp.round(half_s[...] * jnp.tanh(zt))
    out_ref[...] = jnp.dot(wo_s[...], q, preferred_element_type=f32)
    codes_t = jnp.dot(pa_s[...], q, preferred_element_type=f32)
    codes_ref[...] = (codes_t + ca_s[...]).astype(jnp.int32)


def kernel(x, W_enc_lips, W_enc_exp, W_enc_rest, W_enc_rot,
           W_dec_lips, W_dec_exp, W_dec_rest, W_dec_rot):
    B, T, F = x.shape
    f32 = jnp.float32

    xa = x.transpose(2, 0, 1).reshape(F * B, T)
    wlt, wet = W_enc_lips.T, W_enc_exp.T
    wrt, wot = W_enc_rest.T, W_enc_rot.T

    half = jnp.array([[3.5]] * 10 + [[2.0]] * 6 + [[3.0]] * 4, f32)
    pt = jnp.array(
        [[float(_L_LIPS ** i) for i in range(5)] + [0.0] * 15,
         [0.0] * 5 + [float(_L_EXP ** i) for i in range(5)] + [0.0] * 10,
         [0.0] * 10 + [float(_L_REST ** i) for i in range(6)] + [0.0] * 4,
         [0.0] * 16 + [float(_L_ROT ** i) for i in range(4)]], f32)
    c_vec = jnp.array(
        [[_OFF_LIPS + 3.0 * sum(_L_LIPS ** i for i in range(5))],
         [_OFF_EXP + 3.0 * sum(_L_EXP ** i for i in range(5))],
         [_OFF_REST + 2.0 * sum(_L_REST ** i for i in range(6))],
         [_OFF_ROT + 3.0 * sum(_L_ROT ** i for i in range(4))]], f32)

    full = lambda shape: pl.BlockSpec(shape, lambda b: tuple(0 for _ in shape))
    outa, codesa = pl.pallas_call(
        _body,
        grid=(T // _TB,),
        in_specs=[
            pl.BlockSpec((F * B, _TB), lambda b: (0, b)),
            full((5, 63)), full((5, 63)), full((6, 130)), full((4, 12)),
            full((5, 15)), full((5, 48)), full((6, 132)), full((4, 10)),
            full((_DQ, 1)), full((4, _DQ)), full((4, 1)),
        ],
        out_specs=[
            pl.BlockSpec((F * B, _TB), lambda b: (0, b)),
            pl.BlockSpec((4 * B, _TB), lambda b: (0, b)),
        ],
        out_shape=[
            jax.ShapeDtypeStruct((F * B, T), f32),
            jax.ShapeDtypeStruct((4 * B, T), jnp.int32),
        ],
        scratch_shapes=[
            pltpu.VMEM((_DQ * B, F * B), f32),
            pltpu.VMEM((F * B, _DQ * B), f32),
            pltpu.VMEM((4 * B, _DQ * B), f32),
            pltpu.VMEM((_DQ * B, 1), f32),
            pltpu.VMEM((4 * B, 1), f32),
        ],
        compiler_params=pltpu.CompilerParams(
            dimension_semantics=("arbitrary",),
        ),
    )(xa, wlt, wet, wrt, wot, W_dec_lips, W_dec_exp, W_dec_rest, W_dec_rot,
      half, pt, c_vec)

    out = outa.reshape(F, B, T).transpose(1, 2, 0)
    codes = codesa.reshape(4, B, T)
    return out, codes
